# baseline (speedup 1.0000x reference)
import functools

import jax
import jax.numpy as jnp
from jax import lax
from jax.experimental import pallas as pl
from jax.experimental.pallas import tpu as pltpu
from jax.experimental.pallas import tpu_sc as plsc

B = 128
D = 768
F = 65536
K = 64
BF = 1024
NB = F // BF
CW = 2048
NCH = F // CW
NBLK = 512
NLVL = 6
GB = 128
NGB = F // GB
BCAP = 128

_I32_MIN = -2147483648
_MASK31 = 0x7FFFFFFF
_NEG_INF = float("-inf")


def _mono_key(x):
    b = lax.bitcast_convert_type(x, jnp.int32)
    return jnp.where(b < 0, b ^ _MASK31, b)


def _unmono(k):
    b = jnp.where(k < 0, k ^ _MASK31, k)
    return lax.bitcast_convert_type(b, jnp.float32)


def _tc_body(embed_ref, bias_ref, wenc_ref,
             scores_ref, m1_ref, tq_ref, mq_ref,
             sc_ref, x_ref, mt_ref, m1t_ref):
    i = pl.program_id(0)

    @pl.when(i == 0)
    def _init():
        x_ref[...] = embed_ref[...] - bias_ref[...]
        mt_ref[...] = jnp.full((B, NLVL * NBLK), _NEG_INF, jnp.float32)

    s = lax.dot_general(x_ref[...], wenc_ref[...],
                        (((1,), (1,)), ((), ())),
                        preferred_element_type=jnp.float32)
    sc_ref[:, pl.ds(pl.multiple_of(i * BF, BF), BF)] = s
    scores_ref[...] = s

    st = lax.transpose(s, (1, 0))
    m1s = [jnp.max(st[blk * GB:(blk + 1) * GB, :], axis=0, keepdims=True)
           for blk in range(BF // GB)]
    m1t_ref[pl.ds(pl.multiple_of(i * (BF // GB), 8), BF // GB), :] = (
        jnp.concatenate(m1s, axis=0))

    for sub in range(BF // NBLK):
        v = s[:, sub * NBLK:(sub + 1) * NBLK]
        for l in range(NLVL):
            cur = mt_ref[:, l * NBLK:(l + 1) * NBLK]
            hi = jnp.maximum(cur, v)
            v = jnp.minimum(cur, v)
            mt_ref[:, l * NBLK:(l + 1) * NBLK] = hi

    @pl.when(i == NB - 1)
    def _select():
        def count_ge(cand):
            def chunk(c, acc):
                sv = sc_ref[:, pl.ds(pl.multiple_of(c * CW, CW), CW)]
                hit = (_mono_key(sv) >= cand).astype(jnp.int32)
                return acc + jnp.sum(hit, axis=1, keepdims=True)
            return lax.fori_loop(0, NCH, chunk, jnp.zeros((B, 1), jnp.int32))

        def count_ge2(cand):
            def chunk(c, accs):
                a1, a2 = accs
                sv = sc_ref[:, pl.ds(pl.multiple_of(c * CW, CW), CW)]
                k = _mono_key(sv)
                h1 = (k >= cand).astype(jnp.int32)
                h2 = (k >= cand + 1).astype(jnp.int32)
                return (a1 + jnp.sum(h1, axis=1, keepdims=True),
                        a2 + jnp.sum(h2, axis=1, keepdims=True))
            z = jnp.zeros((B, 1), jnp.int32)
            return lax.fori_loop(0, NCH, chunk, (z, z))

        mkeys = _mono_key(lax.transpose(mt_ref[...], (1, 0)))

        def mcount(cand):
            return jnp.sum((mkeys >= cand).astype(jnp.int32),
                           axis=0, keepdims=True)

        def ms_iter(j, t):
            cand = t + (jnp.int32(1) << (30 - j).astype(jnp.int32))
            return jnp.where(mcount(cand) >= K, cand, t)

        tm0 = jnp.where(mcount(jnp.zeros((1, B), jnp.int32)) >= K,
                        0, _I32_MIN).astype(jnp.int32)
        tm = lax.fori_loop(0, 31, ms_iter, tm0)

        def to_col(row):
            return lax.transpose(jnp.broadcast_to(row, (B, B)), (1, 0))[:, 0:1]

        tm_col = to_col(tm)
        mc_col = to_col(mcount(tm))
        cge_tm, cgt_tm = count_ge2(tm_col)
        ok = jnp.sum(jnp.where(cge_tm == mc_col, 0, 1)) == 0
        any_tie = jnp.sum(jnp.where(cge_tm != K, 1, 0)) > 0

        tq_ref[...] = jnp.broadcast_to(_unmono(tm_col), (B, 128))
        mq_ref[...] = jnp.full((B, 128), F, jnp.int32)
        m1_ref[...] = lax.transpose(m1t_ref[...], (1, 0))

        @pl.when(jnp.logical_not(ok) | any_tie)
        def _slow_exact():
            def bs_iter(j, t):
                cand = t + (jnp.int32(1) << (30 - j).astype(jnp.int32))
                return jnp.where(count_ge(cand) >= K, cand, t)

            t0 = jnp.where(count_ge(jnp.zeros((B, 1), jnp.int32)) >= K,
                           0, _I32_MIN).astype(jnp.int32)
            tf = lax.fori_loop(0, 31, bs_iter, t0)
            _, cgt_f = count_ge2(tf)
            tq_ref[...] = jnp.broadcast_to(_unmono(tf), (B, 128))
            need = K - cgt_f
            tf32 = _unmono(tf)

            def count_eq_lt(cand):
                def chunk(c, acc):
                    sv = sc_ref[:, pl.ds(pl.multiple_of(c * CW, CW), CW)]
                    idx = (lax.broadcasted_iota(jnp.int32, (B, CW), 1)
                           + c * CW)
                    hit = ((sv == tf32) & (idx < cand)).astype(jnp.int32)
                    return acc + jnp.sum(hit, axis=1, keepdims=True)
                return lax.fori_loop(0, NCH, chunk,
                                     jnp.zeros((B, 1), jnp.int32))

            def m_iter(j, m):
                cand = m + (jnp.int32(1) << (16 - j).astype(jnp.int32))
                c = count_eq_lt(cand)
                return jnp.where(c < need, cand, m)

            m = lax.fori_loop(0, 17, m_iter, jnp.zeros((B, 1), jnp.int32))
            mq_ref[...] = jnp.broadcast_to(m + 1, (B, 128))


def _tc_stage(embed, W_enc, bias2):
    return pl.pallas_call(
        _tc_body,
        grid=(NB,),
        in_specs=[
            pl.BlockSpec((B, D), lambda i: (0, 0)),
            pl.BlockSpec((1, D), lambda i: (0, 0)),
            pl.BlockSpec((BF, D), lambda i: (i, 0)),
        ],
        out_specs=[
            pl.BlockSpec((B, BF), lambda i: (0, i)),
            pl.BlockSpec((B, NGB), lambda i: (0, 0)),
            pl.BlockSpec((B, 128), lambda i: (0, 0)),
            pl.BlockSpec((B, 128), lambda i: (0, 0)),
        ],
        out_shape=[
            jax.ShapeDtypeStruct((B, F), jnp.float32),
            jax.ShapeDtypeStruct((B, NGB), jnp.float32),
            jax.ShapeDtypeStruct((B, 128), jnp.float32),
            jax.ShapeDtypeStruct((B, 128), jnp.int32),
        ],
        scratch_shapes=[
            pltpu.VMEM((B, F), jnp.float32),
            pltpu.VMEM((B, D), jnp.float32),
            pltpu.VMEM((B, NLVL * NBLK), jnp.float32),
            pltpu.VMEM((NGB, B), jnp.float32),
        ],
    )(embed, bias2, W_enc)


def _iota16():
    return lax.iota(jnp.int32, 16)


def _sc_decode(scores2d, m1, tq, mq, W_dec, bias2):
    info = plsc.get_sparse_core_info()
    nc = info.num_cores
    ns = info.num_subcores
    rows_per_w = B // (nc * ns)

    mesh = plsc.VectorSubcoreMesh(core_axis_name="c", subcore_axis_name="s")

    @functools.partial(
        pl.kernel, mesh=mesh,
        out_type=jax.ShapeDtypeStruct((B, D), jnp.float32),
        compiler_params=pltpu.CompilerParams(needs_layout_passes=False),
        scratch_types=[
            pltpu.VMEM((NGB,), jnp.float32),
            pltpu.VMEM((16,), jnp.float32),
            pltpu.VMEM((16,), jnp.int32),
            pltpu.VMEM((BCAP,), jnp.int32),
            pltpu.VMEM((BCAP, GB), jnp.float32),
            pltpu.VMEM((K,), jnp.int32),
            pltpu.VMEM((K,), jnp.float32),
            pltpu.VMEM((K, D), jnp.float32),
            pltpu.VMEM((D,), jnp.float32),
            pltpu.VMEM((D,), jnp.float32),
            pltpu.SemaphoreType.DMA,
        ],
    )
    def k(scores_hbm, m1_hbm, tq_hbm, mq_hbm, wdec_hbm, bias_hbm, out_hbm,
          m1_v, t_v, m_v, gidx_v, gath_v, fsel_v, wsel_v, vrows_v,
          acc_v, bias_v, sem):
        wid = lax.axis_index("s") * nc + lax.axis_index("c")
        pltpu.sync_copy(bias_hbm.at[0], bias_v)

        def do_row(r, carry):
            b = wid * rows_per_w + r
            pltpu.sync_copy(m1_hbm.at[b], m1_v)
            pltpu.sync_copy(tq_hbm.at[b, pl.ds(0, 16)], t_v)
            pltpu.sync_copy(mq_hbm.at[b, pl.ds(0, 16)], m_v)
            t = t_v[...]
            m = m_v[...]
            zero = jnp.zeros((16,), jnp.int32)

            for c in range(BCAP // 16):
                gidx_v[pl.ds(c * 16, 16)] = zero + b * NGB

            def blk_scan(c, run):
                v = m1_v[pl.ds(c * 16, 16)]
                blkid = c * 16 + _iota16()
                cand = (v > t) | ((v == t) & (blkid * GB < m))
                ci = jnp.where(cand, 1, 0)
                pos = run + plsc.cumsum(ci) - 1
                pos = jnp.minimum(pos, BCAP - 1)
                plsc.store_scatter(gidx_v, [pos], b * NGB + blkid, mask=cand)
                return run + jnp.sum(ci)
            nblk = lax.fori_loop(0, NGB // 16, blk_scan, zero)

            pltpu.async_copy(scores_hbm.at[gidx_v], gath_v, sem).wait()

            def ext_j(j, run2):
                base = (plsc.load_gather(gidx_v, [zero + j])
                        - b * NGB) * GB
                valid = (zero + j) < nblk

                def ext_q(q, run2q):
                    v = gath_v[j, pl.ds(q * 16, 16)]
                    gidx16 = base + q * 16 + _iota16()
                    sel = ((v > t) | ((v == t) & (gidx16 < m))) & valid
                    si = jnp.where(sel, 1, 0)
                    pos = run2q + plsc.cumsum(si) - 1
                    pos = jnp.minimum(pos, K - 1)
                    plsc.store_scatter(fsel_v, [pos], gidx16, mask=sel)
                    plsc.store_scatter(wsel_v, [pos], v, mask=sel)
                    return run2q + jnp.sum(si)
                return lax.fori_loop(0, GB // 16, ext_q, run2)
            lax.fori_loop(0, BCAP, ext_j, zero)

            pltpu.async_copy(wdec_hbm.at[fsel_v], vrows_v, sem).wait()
            for d in range(D // 16):
                acc_v[pl.ds(d * 16, 16)] = bias_v[pl.ds(d * 16, 16)]

            def dec_t(tt, carry2):
                w = plsc.load_gather(wsel_v, [zero + tt])
                for d in range(D // 16):
                    plsc.addupdate(acc_v.at[pl.ds(d * 16, 16)],
                                   w * vrows_v[tt, pl.ds(d * 16, 16)])
                return carry2
            lax.fori_loop(0, K, dec_t, 0)

            ssq = jnp.zeros((16,), jnp.float32)
            for d in range(D // 16):
                a = acc_v[pl.ds(d * 16, 16)]
                ssq = ssq + a * a
            stot = jnp.sum(ssq)
            sv = jnp.zeros((16,), jnp.float32) + stot
            yb = lax.bitcast_convert_type(sv, jnp.int32)
            y = lax.bitcast_convert_type(
                0x5F3759DF - lax.shift_right_logical(yb, 1), jnp.float32)
            for _ in range(3):
                y = y * (1.5 - 0.5 * sv * y * y)
            norm = jnp.maximum(sv * y, 1e-12)
            inv = 1.0 / norm
            for d in range(D // 16):
                acc_v[pl.ds(d * 16, 16)] = acc_v[pl.ds(d * 16, 16)] * inv
            pltpu.sync_copy(acc_v, out_hbm.at[b])
            return carry

        lax.fori_loop(0, rows_per_w, do_row, 0)

    return k(scores2d, m1, tq, mq, W_dec, bias2)


def kernel(embed, W_enc, W_dec, bias):
    bias2 = bias.reshape(1, D)
    scores, m1, tq, mq = _tc_stage(embed, W_enc, bias2)
    scores2d = scores.reshape(B * NGB, GB)
    return _sc_decode(scores2d, m1, tq, mq, W_dec, bias2)


# speedup vs baseline: 1.3191x; 1.3191x over previous
import functools

import jax
import jax.numpy as jnp
from jax import lax
from jax.experimental import pallas as pl
from jax.experimental.pallas import tpu as pltpu
from jax.experimental.pallas import tpu_sc as plsc

B = 128
D = 768
F = 65536
K = 64
BF = 1024
NB = F // BF
CW = 2048
NCH = F // CW
NBLK = 512
NLVL = 6
GB = 128
NGB = F // GB
BCAP = 128

_I32_MIN = -2147483648
_MASK31 = 0x7FFFFFFF
_NEG_INF = float("-inf")


def _mono_key(x):
    b = lax.bitcast_convert_type(x, jnp.int32)
    return jnp.where(b < 0, b ^ _MASK31, b)


def _unmono(k):
    b = jnp.where(k < 0, k ^ _MASK31, k)
    return lax.bitcast_convert_type(b, jnp.float32)


def _tc_body(embed_ref, bias_ref, wenc_ref,
             scores_ref, m1_ref, tq_ref, mq_ref,
             sc_ref, x_ref, mt_ref, m1t_ref):
    i = pl.program_id(0)

    @pl.when(i == 0)
    def _init():
        x_ref[...] = embed_ref[...] - bias_ref[...]
        mt_ref[...] = jnp.full((B, NLVL * NBLK), _NEG_INF, jnp.float32)

    s = lax.dot_general(x_ref[...], wenc_ref[...],
                        (((1,), (1,)), ((), ())),
                        preferred_element_type=jnp.float32)
    sc_ref[:, pl.ds(pl.multiple_of(i * BF, BF), BF)] = s
    scores_ref[...] = s

    st = lax.transpose(s, (1, 0))
    m1s = [jnp.max(st[blk * GB:(blk + 1) * GB, :], axis=0, keepdims=True)
           for blk in range(BF // GB)]
    m1t_ref[pl.ds(pl.multiple_of(i * (BF // GB), 8), BF // GB), :] = (
        jnp.concatenate(m1s, axis=0))

    for sub in range(BF // NBLK):
        v = s[:, sub * NBLK:(sub + 1) * NBLK]
        for l in range(NLVL):
            cur = mt_ref[:, l * NBLK:(l + 1) * NBLK]
            hi = jnp.maximum(cur, v)
            v = jnp.minimum(cur, v)
            mt_ref[:, l * NBLK:(l + 1) * NBLK] = hi

    @pl.when(i == NB - 1)
    def _select():
        def count_ge(cand):
            def chunk(c, acc):
                sv = sc_ref[:, pl.ds(pl.multiple_of(c * CW, CW), CW)]
                hit = (_mono_key(sv) >= cand).astype(jnp.int32)
                return acc + jnp.sum(hit, axis=1, keepdims=True)
            return lax.fori_loop(0, NCH, chunk, jnp.zeros((B, 1), jnp.int32))

        def count_ge2(cand):
            def chunk(c, accs):
                a1, a2 = accs
                sv = sc_ref[:, pl.ds(pl.multiple_of(c * CW, CW), CW)]
                k = _mono_key(sv)
                h1 = (k >= cand).astype(jnp.int32)
                h2 = (k >= cand + 1).astype(jnp.int32)
                return (a1 + jnp.sum(h1, axis=1, keepdims=True),
                        a2 + jnp.sum(h2, axis=1, keepdims=True))
            z = jnp.zeros((B, 1), jnp.int32)
            return lax.fori_loop(0, NCH, chunk, (z, z))

        mkeys = _mono_key(lax.transpose(mt_ref[...], (1, 0)))

        def mcount(cand):
            return jnp.sum((mkeys >= cand).astype(jnp.int32),
                           axis=0, keepdims=True)

        def ms_iter(j, t):
            cand = t + (jnp.int32(1) << (30 - j).astype(jnp.int32))
            return jnp.where(mcount(cand) >= K, cand, t)

        tm0 = jnp.where(mcount(jnp.zeros((1, B), jnp.int32)) >= K,
                        0, _I32_MIN).astype(jnp.int32)
        tm = lax.fori_loop(0, 31, ms_iter, tm0)

        def to_col(row):
            return lax.transpose(jnp.broadcast_to(row, (B, B)), (1, 0))[:, 0:1]

        tm_col = to_col(tm)
        mc_col = to_col(mcount(tm))
        cge_tm, cgt_tm = count_ge2(tm_col)
        ok = jnp.sum(jnp.where(cge_tm == mc_col, 0, 1)) == 0
        any_tie = jnp.sum(jnp.where(cge_tm != K, 1, 0)) > 0

        tq_ref[...] = jnp.broadcast_to(_unmono(tm_col), (B, 128))
        mq_ref[...] = jnp.full((B, 128), F, jnp.int32)
        m1_ref[...] = lax.transpose(m1t_ref[...], (1, 0))

        @pl.when(jnp.logical_not(ok) | any_tie)
        def _slow_exact():
            def bs_iter(j, t):
                cand = t + (jnp.int32(1) << (30 - j).astype(jnp.int32))
                return jnp.where(count_ge(cand) >= K, cand, t)

            t0 = jnp.where(count_ge(jnp.zeros((B, 1), jnp.int32)) >= K,
                           0, _I32_MIN).astype(jnp.int32)
            tf = lax.fori_loop(0, 31, bs_iter, t0)
            _, cgt_f = count_ge2(tf)
            tq_ref[...] = jnp.broadcast_to(_unmono(tf), (B, 128))
            need = K - cgt_f
            tf32 = _unmono(tf)

            def count_eq_lt(cand):
                def chunk(c, acc):
                    sv = sc_ref[:, pl.ds(pl.multiple_of(c * CW, CW), CW)]
                    idx = (lax.broadcasted_iota(jnp.int32, (B, CW), 1)
                           + c * CW)
                    hit = ((sv == tf32) & (idx < cand)).astype(jnp.int32)
                    return acc + jnp.sum(hit, axis=1, keepdims=True)
                return lax.fori_loop(0, NCH, chunk,
                                     jnp.zeros((B, 1), jnp.int32))

            def m_iter(j, m):
                cand = m + (jnp.int32(1) << (16 - j).astype(jnp.int32))
                c = count_eq_lt(cand)
                return jnp.where(c < need, cand, m)

            m = lax.fori_loop(0, 17, m_iter, jnp.zeros((B, 1), jnp.int32))
            mq_ref[...] = jnp.broadcast_to(m + 1, (B, 128))


def _tc_stage(embed, W_enc, bias2):
    return pl.pallas_call(
        _tc_body,
        grid=(NB,),
        in_specs=[
            pl.BlockSpec((B, D), lambda i: (0, 0)),
            pl.BlockSpec((1, D), lambda i: (0, 0)),
            pl.BlockSpec((BF, D), lambda i: (i, 0)),
        ],
        out_specs=[
            pl.BlockSpec((B, BF), lambda i: (0, i)),
            pl.BlockSpec((B, NGB), lambda i: (0, 0)),
            pl.BlockSpec((B, 128), lambda i: (0, 0)),
            pl.BlockSpec((B, 128), lambda i: (0, 0)),
        ],
        out_shape=[
            jax.ShapeDtypeStruct((B, F), jnp.float32),
            jax.ShapeDtypeStruct((B, NGB), jnp.float32),
            jax.ShapeDtypeStruct((B, 128), jnp.float32),
            jax.ShapeDtypeStruct((B, 128), jnp.int32),
        ],
        scratch_shapes=[
            pltpu.VMEM((B, F), jnp.float32),
            pltpu.VMEM((B, D), jnp.float32),
            pltpu.VMEM((B, NLVL * NBLK), jnp.float32),
            pltpu.VMEM((NGB, B), jnp.float32),
        ],
    )(embed, bias2, W_enc)


def _iota16():
    return lax.iota(jnp.int32, 16)


def _sc_decode(scores2d, m1, tq, mq, W_dec, bias2):
    info = plsc.get_sparse_core_info()
    nc = info.num_cores
    ns = info.num_subcores
    rows_per_w = B // (nc * ns)

    mesh = plsc.VectorSubcoreMesh(core_axis_name="c", subcore_axis_name="s")

    @functools.partial(
        pl.kernel, mesh=mesh,
        out_type=jax.ShapeDtypeStruct((B, D), jnp.float32),
        compiler_params=pltpu.CompilerParams(needs_layout_passes=False),
        scratch_types=[
            pltpu.VMEM((NGB,), jnp.float32),
            pltpu.VMEM((16,), jnp.float32),
            pltpu.VMEM((16,), jnp.int32),
            pltpu.VMEM((BCAP,), jnp.int32),
            pltpu.VMEM((BCAP, GB), jnp.float32),
            pltpu.VMEM((K,), jnp.int32),
            pltpu.VMEM((K,), jnp.float32),
            pltpu.VMEM((K, D), jnp.float32),
            pltpu.VMEM((D,), jnp.float32),
            pltpu.VMEM((D,), jnp.float32),
            pltpu.SemaphoreType.DMA,
        ],
    )
    def k(scores_hbm, m1_hbm, tq_hbm, mq_hbm, wdec_hbm, bias_hbm, out_hbm,
          m1_v, t_v, m_v, gidx_v, gath_v, fsel_v, wsel_v, vrows_v,
          acc_v, bias_v, sem):
        wid = lax.axis_index("s") * nc + lax.axis_index("c")
        pltpu.sync_copy(bias_hbm.at[0], bias_v)

        def do_row(r, carry):
            b = wid * rows_per_w + r
            pltpu.sync_copy(m1_hbm.at[b], m1_v)
            pltpu.sync_copy(tq_hbm.at[b, pl.ds(0, 16)], t_v)
            pltpu.sync_copy(mq_hbm.at[b, pl.ds(0, 16)], m_v)
            t = t_v[...]
            m = m_v[...]
            zero = jnp.zeros((16,), jnp.int32)

            for c in range(BCAP // 16):
                gidx_v[pl.ds(c * 16, 16)] = zero + b * NGB

            def blk_scan(c, run):
                v = m1_v[pl.ds(c * 16, 16)]
                blkid = c * 16 + _iota16()
                cand = (v > t) | ((v == t) & (blkid * GB < m))
                ci = jnp.where(cand, 1, 0)
                pos = run + plsc.cumsum(ci) - 1
                pos = jnp.minimum(pos, BCAP - 1)
                plsc.store_scatter(gidx_v, [pos], b * NGB + blkid, mask=cand)
                return run + jnp.sum(ci)
            nblk = lax.fori_loop(0, NGB // 16, blk_scan, zero)

            pltpu.async_copy(scores_hbm.at[gidx_v], gath_v, sem).wait()

            nblk_s = jnp.max(jnp.minimum(nblk, BCAP))

            def ext_j(j, run2):
                base = (plsc.load_gather(gidx_v, [zero + j])
                        - b * NGB) * GB

                def ext_q(q, run2q):
                    v = gath_v[j, pl.ds(q * 16, 16)]
                    gidx16 = base + q * 16 + _iota16()
                    sel = (v > t) | ((v == t) & (gidx16 < m))
                    si = jnp.where(sel, 1, 0)
                    pos = run2q + plsc.cumsum(si) - 1
                    pos = jnp.minimum(pos, K - 1)
                    plsc.store_scatter(fsel_v, [pos], gidx16, mask=sel)
                    plsc.store_scatter(wsel_v, [pos], v, mask=sel)
                    return run2q + jnp.sum(si)
                return lax.fori_loop(0, GB // 16, ext_q, run2)
            lax.fori_loop(0, nblk_s, ext_j, zero)

            pltpu.async_copy(wdec_hbm.at[fsel_v], vrows_v, sem).wait()

            def dec_t(tt, accs):
                w = plsc.load_gather(wsel_v, [zero + tt])
                return tuple(
                    accs[d] + w * vrows_v[tt, pl.ds(d * 16, 16)]
                    for d in range(D // 16))
            acc0 = tuple(bias_v[pl.ds(d * 16, 16)] for d in range(D // 16))
            accs = lax.fori_loop(0, K, dec_t, acc0)
            for d in range(D // 16):
                acc_v[pl.ds(d * 16, 16)] = accs[d]

            ssq = jnp.zeros((16,), jnp.float32)
            for d in range(D // 16):
                ssq = ssq + accs[d] * accs[d]
            stot = jnp.sum(ssq)
            sv = jnp.zeros((16,), jnp.float32) + stot
            yb = lax.bitcast_convert_type(sv, jnp.int32)
            y = lax.bitcast_convert_type(
                0x5F3759DF - lax.shift_right_logical(yb, 1), jnp.float32)
            for _ in range(3):
                y = y * (1.5 - 0.5 * sv * y * y)
            norm = jnp.maximum(sv * y, 1e-12)
            inv = 1.0 / norm
            for d in range(D // 16):
                acc_v[pl.ds(d * 16, 16)] = acc_v[pl.ds(d * 16, 16)] * inv
            pltpu.sync_copy(acc_v, out_hbm.at[b])
            return carry

        lax.fori_loop(0, rows_per_w, do_row, 0)

    return k(scores2d, m1, tq, mq, W_dec, bias2)


def kernel(embed, W_enc, W_dec, bias):
    bias2 = bias.reshape(1, D)
    scores, m1, tq, mq = _tc_stage(embed, W_enc, bias2)
    scores2d = scores.reshape(B * NGB, GB)
    return _sc_decode(scores2d, m1, tq, mq, W_dec, bias2)


# speedup vs baseline: 1.4283x; 1.0827x over previous
import functools

import jax
import jax.numpy as jnp
from jax import lax
from jax.experimental import pallas as pl
from jax.experimental.pallas import tpu as pltpu
from jax.experimental.pallas import tpu_sc as plsc

B = 128
D = 768
F = 65536
K = 64
BF = 2048
NB = F // BF
CW = 2048
NCH = F // CW
NBLK = 512
NLVL = 6
GB = 128
NGB = F // GB
BCAP = 128

_I32_MIN = -2147483648
_MASK31 = 0x7FFFFFFF
_NEG_INF = float("-inf")


def _mono_key(x):
    b = lax.bitcast_convert_type(x, jnp.int32)
    return jnp.where(b < 0, b ^ _MASK31, b)


def _unmono(k):
    b = jnp.where(k < 0, k ^ _MASK31, k)
    return lax.bitcast_convert_type(b, jnp.float32)


def _tc_body(embed_ref, bias_ref, wenc_ref,
             scores_ref, m1_ref, tq_ref, mq_ref,
             sc_ref, x_ref, mt_ref, m1t_ref):
    i = pl.program_id(0)

    @pl.when(i == 0)
    def _init():
        x_ref[...] = embed_ref[...] - bias_ref[...]
        mt_ref[...] = jnp.full((B, NLVL * NBLK), _NEG_INF, jnp.float32)

    s = lax.dot_general(x_ref[...], wenc_ref[...],
                        (((1,), (1,)), ((), ())),
                        preferred_element_type=jnp.float32)
    sc_ref[:, pl.ds(pl.multiple_of(i * BF, BF), BF)] = s
    scores_ref[...] = s

    st = lax.transpose(s, (1, 0))
    m1s = [jnp.max(st[blk * GB:(blk + 1) * GB, :], axis=0, keepdims=True)
           for blk in range(BF // GB)]
    m1t_ref[pl.ds(pl.multiple_of(i * (BF // GB), 8), BF // GB), :] = (
        jnp.concatenate(m1s, axis=0))

    for sub in range(BF // NBLK):
        v = s[:, sub * NBLK:(sub + 1) * NBLK]
        for l in range(NLVL):
            cur = mt_ref[:, l * NBLK:(l + 1) * NBLK]
            hi = jnp.maximum(cur, v)
            v = jnp.minimum(cur, v)
            mt_ref[:, l * NBLK:(l + 1) * NBLK] = hi

    @pl.when(i == NB - 1)
    def _select():
        def count_ge(cand):
            def chunk(c, acc):
                sv = sc_ref[:, pl.ds(pl.multiple_of(c * CW, CW), CW)]
                hit = (_mono_key(sv) >= cand).astype(jnp.int32)
                return acc + jnp.sum(hit, axis=1, keepdims=True)
            return lax.fori_loop(0, NCH, chunk, jnp.zeros((B, 1), jnp.int32))

        def count_ge2(cand):
            def chunk(c, accs):
                a1, a2 = accs
                sv = sc_ref[:, pl.ds(pl.multiple_of(c * CW, CW), CW)]
                k = _mono_key(sv)
                h1 = (k >= cand).astype(jnp.int32)
                h2 = (k >= cand + 1).astype(jnp.int32)
                return (a1 + jnp.sum(h1, axis=1, keepdims=True),
                        a2 + jnp.sum(h2, axis=1, keepdims=True))
            z = jnp.zeros((B, 1), jnp.int32)
            return lax.fori_loop(0, NCH, chunk, (z, z))

        mkeys = _mono_key(lax.transpose(mt_ref[...], (1, 0)))

        def mcount(cand):
            return jnp.sum((mkeys >= cand).astype(jnp.int32),
                           axis=0, keepdims=True)

        def ms_iter(j, t):
            cand = t + (jnp.int32(1) << (30 - j).astype(jnp.int32))
            return jnp.where(mcount(cand) >= K, cand, t)

        tm0 = jnp.where(mcount(jnp.zeros((1, B), jnp.int32)) >= K,
                        0, _I32_MIN).astype(jnp.int32)
        tm = lax.fori_loop(0, 31, ms_iter, tm0)

        def to_col(row):
            return lax.transpose(jnp.broadcast_to(row, (B, B)), (1, 0))[:, 0:1]

        tm_col = to_col(tm)
        mc_col = to_col(mcount(tm))
        cge_tm, cgt_tm = count_ge2(tm_col)
        ok = jnp.sum(jnp.where(cge_tm == mc_col, 0, 1)) == 0
        any_tie = jnp.sum(jnp.where(cge_tm != K, 1, 0)) > 0

        tq_ref[...] = jnp.broadcast_to(_unmono(tm_col), (B, 128))
        mq_ref[...] = jnp.full((B, 128), F, jnp.int32)
        m1_ref[...] = lax.transpose(m1t_ref[...], (1, 0))

        @pl.when(jnp.logical_not(ok) | any_tie)
        def _slow_exact():
            def bs_iter(j, t):
                cand = t + (jnp.int32(1) << (30 - j).astype(jnp.int32))
                return jnp.where(count_ge(cand) >= K, cand, t)

            t0 = jnp.where(count_ge(jnp.zeros((B, 1), jnp.int32)) >= K,
                           0, _I32_MIN).astype(jnp.int32)
            tf = lax.fori_loop(0, 31, bs_iter, t0)
            _, cgt_f = count_ge2(tf)
            tq_ref[...] = jnp.broadcast_to(_unmono(tf), (B, 128))
            need = K - cgt_f
            tf32 = _unmono(tf)

            def count_eq_lt(cand):
                def chunk(c, acc):
                    sv = sc_ref[:, pl.ds(pl.multiple_of(c * CW, CW), CW)]
                    idx = (lax.broadcasted_iota(jnp.int32, (B, CW), 1)
                           + c * CW)
                    hit = ((sv == tf32) & (idx < cand)).astype(jnp.int32)
                    return acc + jnp.sum(hit, axis=1, keepdims=True)
                return lax.fori_loop(0, NCH, chunk,
                                     jnp.zeros((B, 1), jnp.int32))

            def m_iter(j, m):
                cand = m + (jnp.int32(1) << (16 - j).astype(jnp.int32))
                c = count_eq_lt(cand)
                return jnp.where(c < need, cand, m)

            m = lax.fori_loop(0, 17, m_iter, jnp.zeros((B, 1), jnp.int32))
            mq_ref[...] = jnp.broadcast_to(m + 1, (B, 128))


def _tc_stage(embed, W_enc, bias2):
    return pl.pallas_call(
        _tc_body,
        grid=(NB,),
        in_specs=[
            pl.BlockSpec((B, D), lambda i: (0, 0)),
            pl.BlockSpec((1, D), lambda i: (0, 0)),
            pl.BlockSpec((BF, D), lambda i: (i, 0)),
        ],
        out_specs=[
            pl.BlockSpec((B, BF), lambda i: (0, i)),
            pl.BlockSpec((B, NGB), lambda i: (0, 0)),
            pl.BlockSpec((B, 128), lambda i: (0, 0)),
            pl.BlockSpec((B, 128), lambda i: (0, 0)),
        ],
        out_shape=[
            jax.ShapeDtypeStruct((B, F), jnp.float32),
            jax.ShapeDtypeStruct((B, NGB), jnp.float32),
            jax.ShapeDtypeStruct((B, 128), jnp.float32),
            jax.ShapeDtypeStruct((B, 128), jnp.int32),
        ],
        scratch_shapes=[
            pltpu.VMEM((B, F), jnp.float32),
            pltpu.VMEM((B, D), jnp.float32),
            pltpu.VMEM((B, NLVL * NBLK), jnp.float32),
            pltpu.VMEM((NGB, B), jnp.float32),
        ],
    )(embed, bias2, W_enc)


def _iota16():
    return lax.iota(jnp.int32, 16)


def _sc_decode(scores2d, m1, tq, mq, W_dec, bias2):
    info = plsc.get_sparse_core_info()
    nc = info.num_cores
    ns = info.num_subcores
    rows_per_w = B // (nc * ns)

    mesh = plsc.VectorSubcoreMesh(core_axis_name="c", subcore_axis_name="s")

    @functools.partial(
        pl.kernel, mesh=mesh,
        out_type=jax.ShapeDtypeStruct((B, D), jnp.float32),
        compiler_params=pltpu.CompilerParams(needs_layout_passes=False),
        scratch_types=[
            pltpu.VMEM((NGB,), jnp.float32),
            pltpu.VMEM((16,), jnp.float32),
            pltpu.VMEM((16,), jnp.int32),
            pltpu.VMEM((BCAP,), jnp.int32),
            pltpu.VMEM((BCAP, GB), jnp.float32),
            pltpu.VMEM((K,), jnp.int32),
            pltpu.VMEM((K,), jnp.float32),
            pltpu.VMEM((K, D), jnp.float32),
            pltpu.VMEM((D,), jnp.float32),
            pltpu.VMEM((D,), jnp.float32),
            pltpu.SemaphoreType.DMA,
        ],
    )
    def k(scores_hbm, m1_hbm, tq_hbm, mq_hbm, wdec_hbm, bias_hbm, out_hbm,
          m1_v, t_v, m_v, gidx_v, gath_v, fsel_v, wsel_v, vrows_v,
          acc_v, bias_v, sem):
        wid = lax.axis_index("s") * nc + lax.axis_index("c")
        pltpu.sync_copy(bias_hbm.at[0], bias_v)

        def do_row(r, carry):
            b = wid * rows_per_w + r
            pltpu.sync_copy(m1_hbm.at[b], m1_v)
            pltpu.sync_copy(tq_hbm.at[b, pl.ds(0, 16)], t_v)
            pltpu.sync_copy(mq_hbm.at[b, pl.ds(0, 16)], m_v)
            t = t_v[...]
            m = m_v[...]
            zero = jnp.zeros((16,), jnp.int32)

            for c in range(BCAP // 16):
                gidx_v[pl.ds(c * 16, 16)] = zero + b * NGB

            def blk_scan(c, run):
                v = m1_v[pl.ds(c * 16, 16)]
                blkid = c * 16 + _iota16()
                cand = (v > t) | ((v == t) & (blkid * GB < m))
                ci = jnp.where(cand, 1, 0)
                pos = run + plsc.cumsum(ci) - 1
                pos = jnp.minimum(pos, BCAP - 1)
                plsc.store_scatter(gidx_v, [pos], b * NGB + blkid, mask=cand)
                return run + jnp.sum(ci)
            nblk = lax.fori_loop(0, NGB // 16, blk_scan, zero)

            pltpu.async_copy(scores_hbm.at[gidx_v], gath_v, sem).wait()

            nblk_s = jnp.max(jnp.minimum(nblk, BCAP))

            def ext_j(j, run2):
                base = (plsc.load_gather(gidx_v, [zero + j])
                        - b * NGB) * GB

                def ext_q(q, run2q):
                    v = gath_v[j, pl.ds(q * 16, 16)]
                    gidx16 = base + q * 16 + _iota16()
                    sel = (v > t) | ((v == t) & (gidx16 < m))
                    si = jnp.where(sel, 1, 0)
                    pos = run2q + plsc.cumsum(si) - 1
                    pos = jnp.minimum(pos, K - 1)
                    plsc.store_scatter(fsel_v, [pos], gidx16, mask=sel)
                    plsc.store_scatter(wsel_v, [pos], v, mask=sel)
                    return run2q + jnp.sum(si)
                return lax.fori_loop(0, GB // 16, ext_q, run2)
            lax.fori_loop(0, nblk_s, ext_j, zero)

            pltpu.async_copy(wdec_hbm.at[fsel_v], vrows_v, sem).wait()

            def dec_t(tt, accs):
                w = plsc.load_gather(wsel_v, [zero + tt])
                return tuple(
                    accs[d] + w * vrows_v[tt, pl.ds(d * 16, 16)]
                    for d in range(D // 16))
            acc0 = tuple(bias_v[pl.ds(d * 16, 16)] for d in range(D // 16))
            accs = lax.fori_loop(0, K, dec_t, acc0)
            for d in range(D // 16):
                acc_v[pl.ds(d * 16, 16)] = accs[d]

            ssq = jnp.zeros((16,), jnp.float32)
            for d in range(D // 16):
                ssq = ssq + accs[d] * accs[d]
            stot = jnp.sum(ssq)
            sv = jnp.zeros((16,), jnp.float32) + stot
            yb = lax.bitcast_convert_type(sv, jnp.int32)
            y = lax.bitcast_convert_type(
                0x5F3759DF - lax.shift_right_logical(yb, 1), jnp.float32)
            for _ in range(3):
                y = y * (1.5 - 0.5 * sv * y * y)
            norm = jnp.maximum(sv * y, 1e-12)
            inv = 1.0 / norm
            for d in range(D // 16):
                acc_v[pl.ds(d * 16, 16)] = acc_v[pl.ds(d * 16, 16)] * inv
            pltpu.sync_copy(acc_v, out_hbm.at[b])
            return carry

        lax.fori_loop(0, rows_per_w, do_row, 0)

    return k(scores2d, m1, tq, mq, W_dec, bias2)


def kernel(embed, W_enc, W_dec, bias):
    bias2 = bias.reshape(1, D)
    scores, m1, tq, mq = _tc_stage(embed, W_enc, bias2)
    scores2d = scores.reshape(B * NGB, GB)
    return _sc_decode(scores2d, m1, tq, mq, W_dec, bias2)
